# Initial kernel scaffold; baseline (speedup 1.0000x reference)
import jax
import jax.numpy as jnp
from jax.experimental import pallas as pl


def kernel(x, freqs, phases, amps, biases, gate_w, final_scale, final_bias):
    raise NotImplementedError("write your pallas kernel here")


# speedup vs baseline: 1.8488x; 1.8488x over previous
import functools
import math

import jax
import jax.numpy as jnp
from jax.experimental import pallas as pl
from jax.experimental.pallas import tpu as pltpu

N_SWARM = 64
K_ACTIVE = 8
N_BASIS = 256
BLOCK_B = 2048


def _slsn_body(x_ref, freqs_ref, phases_ref, amps_ref, biases_ref, gwt_ref,
               fs_ref, fb_ref, out_ref):
    xb = x_ref[...]
    basis = jnp.sin(xb * freqs_ref[...] + phases_ref[...])
    basis_sum = jnp.sum(basis, axis=1, keepdims=True)
    logits = jnp.dot(basis, gwt_ref[...],
                     preferred_element_type=jnp.float32)

    lane = jax.lax.broadcasted_iota(jnp.int32, logits.shape, 1)
    L = logits
    mask = jnp.zeros(logits.shape, dtype=jnp.bool_)
    m1 = jnp.max(L, axis=1, keepdims=True)
    for k in range(K_ACTIVE):
        m = m1 if k == 0 else jnp.max(L, axis=1, keepdims=True)
        is_max = L == m
        sel = jnp.min(jnp.where(is_max, lane, N_SWARM), axis=1, keepdims=True)
        onehot = lane == sel
        mask = jnp.logical_or(mask, onehot)
        L = jnp.where(onehot, -jnp.inf, L)

    w = jnp.where(mask, jnp.exp(logits - m1), 0.0)
    sum_w = jnp.sum(w, axis=1, keepdims=True)
    sum_wa = jnp.sum(w * amps_ref[...], axis=1, keepdims=True)
    sum_wb = jnp.sum(w * biases_ref[...], axis=1, keepdims=True)
    out = (basis_sum * sum_wa + sum_wb) / sum_w
    out_ref[...] = fs_ref[0, 0] * out + fb_ref[0, 0]


@jax.jit
def kernel(x, freqs, phases, amps, biases, gate_w, final_scale, final_bias):
    B = x.shape[0]
    grid = B // BLOCK_B
    freqs2 = freqs.reshape(1, N_BASIS)
    phases2 = phases.reshape(1, N_BASIS)
    amps2 = amps.reshape(1, N_SWARM)
    biases2 = biases.reshape(1, N_SWARM)
    gwt = gate_w.T
    fs = final_scale.reshape(1, 1)
    fb = final_bias.reshape(1, 1)

    out = pl.pallas_call(
        _slsn_body,
        grid=(grid,),
        in_specs=[
            pl.BlockSpec((BLOCK_B, 1), lambda i: (i, 0)),
            pl.BlockSpec((1, N_BASIS), lambda i: (0, 0)),
            pl.BlockSpec((1, N_BASIS), lambda i: (0, 0)),
            pl.BlockSpec((1, N_SWARM), lambda i: (0, 0)),
            pl.BlockSpec((1, N_SWARM), lambda i: (0, 0)),
            pl.BlockSpec((N_BASIS, N_SWARM), lambda i: (0, 0)),
            pl.BlockSpec((1, 1), lambda i: (0, 0)),
            pl.BlockSpec((1, 1), lambda i: (0, 0)),
        ],
        out_specs=pl.BlockSpec((BLOCK_B, 1), lambda i: (i, 0)),
        out_shape=jax.ShapeDtypeStruct((B, 1), jnp.float32),
    )(x, freqs2, phases2, amps2, biases2, gwt, fs, fb)
    return out


# speedup vs baseline: 11.0255x; 5.9636x over previous
import math

import jax
import jax.numpy as jnp
from jax.experimental import pallas as pl

N_SWARM = 64
K_ACTIVE = 8
N_BASIS = 256
BLOCK_T = 2048

_SIN_C = (6.28318280600484, -41.341420393384425, 81.59616132843102,
          -76.57989422663876, 41.204569115574856, -12.270060617852549)
_MAGIC = 12582912.0


def _slsn_body(x_ref, f2_ref, p2_ref, amps_ref, biases_ref, gw_ref,
               fs_ref, fb_ref, out_ref):
    xb = x_ref[...]
    u = f2_ref[...] * xb + p2_ref[...]
    r = u - jnp.round(u)
    r2 = r * r
    p = jnp.float32(_SIN_C[5])
    for c in _SIN_C[4::-1]:
        p = p * r2 + jnp.float32(c)
    basis = r * p

    basis_sum = jnp.sum(basis, axis=0, keepdims=True)
    logits = jnp.dot(gw_ref[...], basis,
                     preferred_element_type=jnp.float32)

    m1 = jnp.max(logits, axis=0, keepdims=True)
    L = logits
    m = m1
    for _ in range(K_ACTIVE - 1):
        L = jnp.where(L == m, -jnp.inf, L)
        m = jnp.max(L, axis=0, keepdims=True)
    w = jnp.where(logits >= m, jnp.exp(logits - m1), 0.0)
    sum_w = jnp.sum(w, axis=0, keepdims=True)
    sum_wa = jnp.sum(w * amps_ref[...], axis=0, keepdims=True)
    sum_wb = jnp.sum(w * biases_ref[...], axis=0, keepdims=True)
    out = (basis_sum * sum_wa + sum_wb) / sum_w
    out_ref[...] = fs_ref[0, 0] * out + fb_ref[0, 0]


@jax.jit
def kernel(x, freqs, phases, amps, biases, gate_w, final_scale, final_bias):
    B = x.shape[0]
    grid = B // BLOCK_T
    inv2pi = 1.0 / (2.0 * math.pi)
    xr = x.reshape(1, B)
    f2 = (freqs * inv2pi).reshape(N_BASIS, 1)
    p2 = (phases * inv2pi).reshape(N_BASIS, 1)
    amps_c = amps.reshape(N_SWARM, 1)
    biases_c = biases.reshape(N_SWARM, 1)
    fs = final_scale.reshape(1, 1)
    fb = final_bias.reshape(1, 1)

    out = pl.pallas_call(
        _slsn_body,
        grid=(grid,),
        in_specs=[
            pl.BlockSpec((1, BLOCK_T), lambda i: (0, i)),
            pl.BlockSpec((N_BASIS, 1), lambda i: (0, 0)),
            pl.BlockSpec((N_BASIS, 1), lambda i: (0, 0)),
            pl.BlockSpec((N_SWARM, 1), lambda i: (0, 0)),
            pl.BlockSpec((N_SWARM, 1), lambda i: (0, 0)),
            pl.BlockSpec((N_SWARM, N_BASIS), lambda i: (0, 0)),
            pl.BlockSpec((1, 1), lambda i: (0, 0)),
            pl.BlockSpec((1, 1), lambda i: (0, 0)),
        ],
        out_specs=pl.BlockSpec((1, BLOCK_T), lambda i: (0, i)),
        out_shape=jax.ShapeDtypeStruct((1, B), jnp.float32),
    )(xr, f2, p2, amps_c, biases_c, gate_w, fs, fb)
    return out.reshape(B, 1)


# speedup vs baseline: 11.3683x; 1.0311x over previous
import math

import jax
import jax.numpy as jnp
from jax.experimental import pallas as pl

N_SWARM = 64
K_ACTIVE = 8
N_BASIS = 256
BLOCK_T = 4096

_SIN_C = (6.28318280600484, -41.341420393384425, 81.59616132843102,
          -76.57989422663876, 41.204569115574856, -12.270060617852549)


def _slsn_body(x_ref, f2_ref, p2_ref, amps_ref, biases_ref, gw_ref,
               fs_ref, fb_ref, out_ref):
    xb = x_ref[...]
    u = f2_ref[...] * xb + p2_ref[...]
    r = u - jnp.round(u)
    r2 = r * r
    p = jnp.float32(_SIN_C[5])
    for c in _SIN_C[4::-1]:
        p = p * r2 + jnp.float32(c)
    basis = r * p

    basis_sum = jnp.sum(basis, axis=0, keepdims=True)
    logits = jnp.dot(gw_ref[...], basis,
                     preferred_element_type=jnp.float32)

    m1 = jnp.max(logits, axis=0, keepdims=True)
    L = logits
    m = m1
    for _ in range(K_ACTIVE - 1):
        L = jnp.where(L == m, -jnp.inf, L)
        m = jnp.max(L, axis=0, keepdims=True)
    w = jnp.where(logits >= m, jnp.exp(logits - m1), 0.0)
    sum_w = jnp.sum(w, axis=0, keepdims=True)
    sum_wa = jnp.sum(w * amps_ref[...], axis=0, keepdims=True)
    sum_wb = jnp.sum(w * biases_ref[...], axis=0, keepdims=True)
    out = (basis_sum * sum_wa + sum_wb) / sum_w
    out_ref[...] = fs_ref[0, 0] * out + fb_ref[0, 0]


@jax.jit
def kernel(x, freqs, phases, amps, biases, gate_w, final_scale, final_bias):
    B = x.shape[0]
    grid = B // BLOCK_T
    inv2pi = 1.0 / (2.0 * math.pi)
    xr = x.reshape(1, B)
    f2 = (freqs * inv2pi).reshape(N_BASIS, 1)
    p2 = (phases * inv2pi).reshape(N_BASIS, 1)
    amps_c = amps.reshape(N_SWARM, 1)
    biases_c = biases.reshape(N_SWARM, 1)
    fs = final_scale.reshape(1, 1)
    fb = final_bias.reshape(1, 1)

    out = pl.pallas_call(
        _slsn_body,
        grid=(grid,),
        in_specs=[
            pl.BlockSpec((1, BLOCK_T), lambda i: (0, i)),
            pl.BlockSpec((N_BASIS, 1), lambda i: (0, 0)),
            pl.BlockSpec((N_BASIS, 1), lambda i: (0, 0)),
            pl.BlockSpec((N_SWARM, 1), lambda i: (0, 0)),
            pl.BlockSpec((N_SWARM, 1), lambda i: (0, 0)),
            pl.BlockSpec((N_SWARM, N_BASIS), lambda i: (0, 0)),
            pl.BlockSpec((1, 1), lambda i: (0, 0)),
            pl.BlockSpec((1, 1), lambda i: (0, 0)),
        ],
        out_specs=pl.BlockSpec((1, BLOCK_T), lambda i: (0, i)),
        out_shape=jax.ShapeDtypeStruct((1, B), jnp.float32),
    )(xr, f2, p2, amps_c, biases_c, gate_w, fs, fb)
    return out.reshape(B, 1)
